# baseline (speedup 1.0000x reference)
import jax
import jax.numpy as jnp
from jax import lax
from jax.experimental import pallas as pl
from jax.experimental.pallas import tpu as pltpu
from jax.experimental.pallas import tpu_sc as plsc
import functools

NC = 2
NS = 16
NW = NC * NS
L = 16

DIM = 32
CHUNK = 128


def _bpr_body(nchunks, u_hbm, i_hbm, j_hbm, ut_hbm, it_hbm,
              pos_hbm, neg_hbm,
              uidx_v, iidx_v, jidx_v, urows_v, irows_v, jrows_v,
              pos_v, neg_v, sem):
    b_per_w = nchunks * CHUNK
    wid = lax.axis_index("s") * NC + lax.axis_index("c")
    base = wid * b_per_w

    for c in range(nchunks):
        pltpu.sync_copy(u_hbm.at[pl.ds(base + c * CHUNK, CHUNK)], uidx_v.at[c])
        pltpu.sync_copy(i_hbm.at[pl.ds(base + c * CHUNK, CHUNK)], iidx_v.at[c])
        pltpu.sync_copy(j_hbm.at[pl.ds(base + c * CHUNK, CHUNK)], jidx_v.at[c])

    copies = []
    for c in range(nchunks):
        sl = pl.ds(c * CHUNK, CHUNK)
        copies.append(pltpu.async_copy(ut_hbm.at[uidx_v.at[c]], urows_v.at[sl], sem))
        copies.append(pltpu.async_copy(it_hbm.at[iidx_v.at[c]], irows_v.at[sl], sem))
        copies.append(pltpu.async_copy(it_hbm.at[jidx_v.at[c]], jrows_v.at[sl], sem))
    for cp in copies:
        cp.wait()

    iota = lax.iota(jnp.int32, L)

    def group_body(g, _):
        ridx = g * L + iota
        pos = jnp.zeros((L,), jnp.float32)
        neg = jnp.zeros((L,), jnp.float32)
        for d in range(DIM):
            didx = jnp.full((L,), d, jnp.int32)
            ud = plsc.load_gather(urows_v, [ridx, didx])
            vd = plsc.load_gather(irows_v, [ridx, didx])
            wd = plsc.load_gather(jrows_v, [ridx, didx])
            pos = pos + ud * vd
            neg = neg + ud * wd
        pos_v[pl.ds(g * L, L)] = pos
        neg_v[pl.ds(g * L, L)] = neg
        return 0

    lax.fori_loop(0, b_per_w // L, group_body, 0)

    pltpu.sync_copy(pos_v, pos_hbm.at[pl.ds(base, b_per_w)])
    pltpu.sync_copy(neg_v, neg_hbm.at[pl.ds(base, b_per_w)])


def kernel(u, i, j, user_table, item_table):
    batch = u.shape[0]
    assert batch % (NW * CHUNK) == 0
    nchunks = batch // (NW * CHUNK)
    b_per_w = nchunks * CHUNK

    mesh = plsc.VectorSubcoreMesh(core_axis_name="c", subcore_axis_name="s",
                                  num_cores=NC, num_subcores=NS)
    f32 = jnp.float32
    run = pl.kernel(
        functools.partial(_bpr_body, nchunks),
        out_type=(jax.ShapeDtypeStruct((batch,), f32),
                  jax.ShapeDtypeStruct((batch,), f32)),
        mesh=mesh,
        compiler_params=pltpu.CompilerParams(needs_layout_passes=False,
                                             use_tc_tiling_on_sc=False),
        scratch_types=[
            pltpu.VMEM((nchunks, CHUNK), jnp.int32),
            pltpu.VMEM((nchunks, CHUNK), jnp.int32),
            pltpu.VMEM((nchunks, CHUNK), jnp.int32),
            pltpu.VMEM((b_per_w, DIM), f32),
            pltpu.VMEM((b_per_w, DIM), f32),
            pltpu.VMEM((b_per_w, DIM), f32),
            pltpu.VMEM((b_per_w,), f32),
            pltpu.VMEM((b_per_w,), f32),
            pltpu.SemaphoreType.DMA,
        ],
    )
    return run(u.astype(jnp.int32), i.astype(jnp.int32), j.astype(jnp.int32),
               user_table, item_table)
